# Initial kernel scaffold; baseline (speedup 1.0000x reference)
import functools

import jax
import jax.numpy as jnp
from jax import lax
from jax.experimental import pallas as pl
from jax.experimental.pallas import tpu as pltpu
from jax.experimental.pallas import tpu_sc as plsc

_N = 10000
_E = 320000
_D = 128
_H = 128
_C = 64

_NC = 2
_NS = 16
_NW = _NC * _NS
_EPW = _E // _NW
_CH = 80
_NCHUNK = _EPW // _CH
_RPT = _N // _NS
_DEGW = 16

_sc_mesh = plsc.VectorSubcoreMesh(
    core_axis_name="c", subcore_axis_name="s", num_cores=_NC, num_subcores=_NS
)


def _deg_body(src_hbm, dst_hbm, ones_hbm, zeros_hbm, out_hbm,
              src_v, dst_v, ones_v, acc_o, acc_i, sem):
    cid = lax.axis_index("c")
    sid = lax.axis_index("s")
    wid = sid * _NC + cid
    row0 = sid * _RPT
    pltpu.sync_copy(zeros_hbm.at[pl.ds(row0, _RPT)], acc_o.at[pl.ds(row0, _RPT)])
    pltpu.sync_copy(zeros_hbm.at[pl.ds(row0, _RPT)], acc_i.at[pl.ds(row0, _RPT)])
    pltpu.sync_copy(ones_hbm, ones_v)
    plsc.subcore_barrier()

    base0 = wid * _EPW

    def body(i, carry):
        base = base0 + i * _CH
        pltpu.sync_copy(src_hbm.at[pl.ds(base, _CH)], src_v)
        pltpu.sync_copy(dst_hbm.at[pl.ds(base, _CH)], dst_v)
        pltpu.sync_copy(ones_v, acc_o.at[src_v], add=True)
        pltpu.sync_copy(ones_v, acc_i.at[dst_v], add=True)
        return carry

    lax.fori_loop(0, _NCHUNK, body, 0)
    plsc.subcore_barrier()
    pltpu.sync_copy(acc_o.at[pl.ds(row0, _RPT)], out_hbm.at[cid, 0, pl.ds(row0, _RPT)])
    pltpu.sync_copy(acc_i.at[pl.ds(row0, _RPT)], out_hbm.at[cid, 1, pl.ds(row0, _RPT)])


_deg_call = pl.kernel(
    _deg_body,
    out_type=jax.ShapeDtypeStruct((_NC, 2, _N, _DEGW), jnp.float32),
    mesh=_sc_mesh,
    scratch_types=[
        pltpu.VMEM((_CH,), jnp.int32),
        pltpu.VMEM((_CH,), jnp.int32),
        pltpu.VMEM((_CH, _DEGW), jnp.float32),
        pltpu.VMEM_SHARED((_N, _DEGW), jnp.float32),
        pltpu.VMEM_SHARED((_N, _DEGW), jnp.float32),
        pltpu.SemaphoreType.DMA,
    ],
)


def _mp_body(h_hbm, src_hbm, dst_hbm, zeros_hbm, out_hbm,
             src_v, dst_v, rows_v, acc, sem):
    cid = lax.axis_index("c")
    sid = lax.axis_index("s")
    wid = sid * _NC + cid
    row0 = sid * _RPT
    pltpu.sync_copy(zeros_hbm.at[pl.ds(row0, _RPT)], acc.at[pl.ds(row0, _RPT)])
    plsc.subcore_barrier()

    base0 = wid * _EPW

    def body(i, carry):
        base = base0 + i * _CH
        pltpu.sync_copy(src_hbm.at[pl.ds(base, _CH)], src_v)
        pltpu.sync_copy(dst_hbm.at[pl.ds(base, _CH)], dst_v)
        pltpu.async_copy(h_hbm.at[src_v], rows_v, sem).wait()
        pltpu.sync_copy(rows_v, acc.at[dst_v], add=True)
        return carry

    lax.fori_loop(0, _NCHUNK, body, 0)
    plsc.subcore_barrier()
    pltpu.sync_copy(acc.at[pl.ds(row0, _RPT)], out_hbm.at[cid, pl.ds(row0, _RPT)])


_mp_call = pl.kernel(
    _mp_body,
    out_type=jax.ShapeDtypeStruct((_NC, _N, _H), jnp.float32),
    mesh=_sc_mesh,
    scratch_types=[
        pltpu.VMEM((_CH,), jnp.int32),
        pltpu.VMEM((_CH,), jnp.int32),
        pltpu.VMEM((_CH, _H), jnp.float32),
        pltpu.VMEM_SHARED((_N, _H), jnp.float32),
        pltpu.SemaphoreType.DMA,
    ],
)


_RB = 1000
_NGRID = _N // _RB


def _dense1_body(x_ref, w_ref, dego_ref, out_ref):
    ns = lax.rsqrt(jnp.maximum(dego_ref[0] + dego_ref[1], 1.0))
    out_ref[...] = jnp.dot(x_ref[...], w_ref[...],
                           preferred_element_type=jnp.float32) * ns


def _dense1(x, W, dego_p):
    return pl.pallas_call(
        _dense1_body,
        grid=(_NGRID,),
        in_specs=[
            pl.BlockSpec((_RB, _D), lambda j: (j, 0)),
            pl.BlockSpec((_D, _H), lambda j: (0, 0)),
            pl.BlockSpec((2, _RB, 1), lambda j: (0, j, 0)),
        ],
        out_specs=pl.BlockSpec((_RB, _H), lambda j: (j, 0)),
        out_shape=jax.ShapeDtypeStruct((_N, _H), jnp.float32),
    )(x, W, dego_p)


def _dense2_body(p_ref, degi_ref, dego_ref, b1_ref, w_ref, out_ref):
    agg = p_ref[0] + p_ref[1]
    nd = lax.rsqrt(jnp.maximum(degi_ref[0] + degi_ref[1], 1.0))
    h = agg * nd + b1_ref[...]
    h = jnp.where(h > 0, h, jnp.expm1(h))
    ns = lax.rsqrt(jnp.maximum(dego_ref[0] + dego_ref[1], 1.0))
    out_ref[...] = jnp.dot(h, w_ref[...],
                           preferred_element_type=jnp.float32) * ns


def _dense2(p, degi_p, dego_p, b1, W):
    return pl.pallas_call(
        _dense2_body,
        grid=(_NGRID,),
        in_specs=[
            pl.BlockSpec((2, _RB, _H), lambda j: (0, j, 0)),
            pl.BlockSpec((2, _RB, 1), lambda j: (0, j, 0)),
            pl.BlockSpec((2, _RB, 1), lambda j: (0, j, 0)),
            pl.BlockSpec((1, _H), lambda j: (0, 0)),
            pl.BlockSpec((_H, _H), lambda j: (0, 0)),
        ],
        out_specs=pl.BlockSpec((_RB, _H), lambda j: (j, 0)),
        out_shape=jax.ShapeDtypeStruct((_N, _H), jnp.float32),
    )(p, degi_p, dego_p, b1, W)


def _final_body(p_ref, degi_ref, b2_ref, wc_ref, bc_ref, out_ref, acc_ref):
    j = pl.program_id(0)

    @pl.when(j == 0)
    def _():
        acc_ref[...] = jnp.zeros_like(acc_ref)

    agg = p_ref[0] + p_ref[1]
    nd = lax.rsqrt(jnp.maximum(degi_ref[0] + degi_ref[1], 1.0))
    z = agg * nd + b2_ref[...]
    z = jnp.where(z > 0, z, jnp.expm1(z))
    acc_ref[...] += jnp.sum(z, axis=0, keepdims=True)

    @pl.when(j == pl.num_programs(0) - 1)
    def _():
        out_ref[...] = jnp.dot(acc_ref[...] * (1.0 / _N), wc_ref[...],
                               preferred_element_type=jnp.float32) + bc_ref[...]


def _final(p, degi_p, b2, Wc, bc):
    return pl.pallas_call(
        _final_body,
        grid=(_NGRID,),
        in_specs=[
            pl.BlockSpec((2, _RB, _H), lambda j: (0, j, 0)),
            pl.BlockSpec((2, _RB, 1), lambda j: (0, j, 0)),
            pl.BlockSpec((1, _H), lambda j: (0, 0)),
            pl.BlockSpec((_H, _C), lambda j: (0, 0)),
            pl.BlockSpec((1, _C), lambda j: (0, 0)),
        ],
        out_specs=pl.BlockSpec((1, _C), lambda j: (0, 0)),
        out_shape=jax.ShapeDtypeStruct((1, _C), jnp.float32),
        scratch_shapes=[pltpu.VMEM((1, _H), jnp.float32)],
    )(p, degi_p, b2, Wc, bc)


def kernel(features, edge_index, W1, b1, W2, b2, Wc, bc):
    src = edge_index[0]
    dst = edge_index[1]

    ones = jnp.ones((_CH, _DEGW), jnp.float32)
    zeros_deg = jnp.zeros((_N, _DEGW), jnp.float32)
    zeros_h = jnp.zeros((_N, _H), jnp.float32)

    degp = _deg_call(src, dst, ones, zeros_deg)
    dego_p = degp[:, 0, :, 0:1]
    degi_p = degp[:, 1, :, 0:1]

    b1r = b1.reshape(1, _H)
    b2r = b2.reshape(1, _H)
    bcr = bc.reshape(1, _C)

    hs1 = _dense1(features, W1, dego_p)
    p1 = _mp_call(hs1, src, dst, zeros_h)
    hs2 = _dense2(p1, degi_p, dego_p, b1r, W2)
    p2 = _mp_call(hs2, src, dst, zeros_h)
    return _final(p2, degi_p, b2r, Wc, bcr)


# speedup vs baseline: 4.3107x; 4.3107x over previous
import functools

import jax
import jax.numpy as jnp
from jax import lax
from jax.experimental import pallas as pl
from jax.experimental.pallas import tpu as pltpu
from jax.experimental.pallas import tpu_sc as plsc

_N = 10000
_E = 320000
_D = 128
_H = 128
_C = 64

_NC = 2
_NS = 16
_NW = _NC * _NS
_EPW = _E // _NW
_CH = 80
_NCHUNK = _EPW // _CH
_RPT = 624
_TAIL0 = _RPT * _NS
_TAILN = _N - _TAIL0
_DEGW = 16

def _deg_body(src_hbm, dst_hbm, ones_hbm, zeros_hbm, out_hbm,
              src_v, dst_v, ones_v, acc, sem):
    cid = lax.axis_index("c")
    sid = lax.axis_index("s")
    wid = sid * _NC + cid
    row0 = sid * _RPT
    pltpu.sync_copy(zeros_hbm.at[pl.ds(row0, _RPT)], acc.at[pl.ds(row0, _RPT)])

    @pl.when(sid == 0)
    def _():
        pltpu.sync_copy(zeros_hbm.at[pl.ds(_TAIL0, _TAILN)], acc.at[pl.ds(_TAIL0, _TAILN)])

    pltpu.sync_copy(ones_hbm, ones_v)
    plsc.subcore_barrier()

    base0 = wid * _EPW

    def body(i, carry):
        base = base0 + i * _CH
        pltpu.sync_copy(src_hbm.at[pl.ds(base, _CH)], src_v)
        pltpu.sync_copy(dst_hbm.at[pl.ds(base, _CH)], dst_v)
        pltpu.sync_copy(ones_v.at[0], acc.at[src_v], add=True)
        pltpu.sync_copy(ones_v.at[1], acc.at[dst_v], add=True)
        return carry

    lax.fori_loop(0, _NCHUNK, body, 0)
    plsc.subcore_barrier()
    pltpu.sync_copy(acc.at[pl.ds(row0, _RPT)], out_hbm.at[cid, pl.ds(row0, _RPT)])

    @pl.when(sid == 0)
    def _():
        pltpu.sync_copy(acc.at[pl.ds(_TAIL0, _TAILN)], out_hbm.at[cid, pl.ds(_TAIL0, _TAILN)])


@functools.cache
def _deg_call():
    mesh = plsc.VectorSubcoreMesh(
        core_axis_name="c", subcore_axis_name="s",
        num_cores=_NC, num_subcores=_NS,
    )
    return pl.kernel(
        _deg_body,
        out_type=jax.ShapeDtypeStruct((_NC, _N, _H), jnp.float32),
        mesh=mesh,
        scratch_types=[
            pltpu.VMEM((_CH,), jnp.int32),
            pltpu.VMEM((_CH,), jnp.int32),
            pltpu.VMEM((2, _CH, _H), jnp.float32),
            pltpu.VMEM_SHARED((_N, _H), jnp.float32),
            pltpu.SemaphoreType.DMA,
        ],
    )


def _mp_body(h_hbm, src_hbm, dst_hbm, zeros_hbm, out_hbm,
             src_v, dst_v, rows_v, acc, sem):
    cid = lax.axis_index("c")
    sid = lax.axis_index("s")
    wid = sid * _NC + cid
    row0 = sid * _RPT
    pltpu.sync_copy(zeros_hbm.at[pl.ds(row0, _RPT)], acc.at[pl.ds(row0, _RPT)])

    @pl.when(sid == 0)
    def _():
        pltpu.sync_copy(zeros_hbm.at[pl.ds(_TAIL0, _TAILN)], acc.at[pl.ds(_TAIL0, _TAILN)])

    plsc.subcore_barrier()

    base0 = wid * _EPW

    def body(i, carry):
        base = base0 + i * _CH
        pltpu.sync_copy(src_hbm.at[pl.ds(base, _CH)], src_v)
        pltpu.sync_copy(dst_hbm.at[pl.ds(base, _CH)], dst_v)
        pltpu.async_copy(h_hbm.at[src_v], rows_v, sem).wait()
        pltpu.sync_copy(rows_v, acc.at[dst_v], add=True)
        return carry

    lax.fori_loop(0, _NCHUNK, body, 0)
    plsc.subcore_barrier()
    pltpu.sync_copy(acc.at[pl.ds(row0, _RPT)], out_hbm.at[cid, pl.ds(row0, _RPT)])

    @pl.when(sid == 0)
    def _():
        pltpu.sync_copy(acc.at[pl.ds(_TAIL0, _TAILN)], out_hbm.at[cid, pl.ds(_TAIL0, _TAILN)])


@functools.cache
def _mp_call():
    mesh = plsc.VectorSubcoreMesh(
        core_axis_name="c", subcore_axis_name="s",
        num_cores=_NC, num_subcores=_NS,
    )
    return pl.kernel(
        _mp_body,
        out_type=jax.ShapeDtypeStruct((_NC, _N, _H), jnp.float32),
        mesh=mesh,
        scratch_types=[
            pltpu.VMEM((_CH,), jnp.int32),
            pltpu.VMEM((_CH,), jnp.int32),
            pltpu.VMEM((_CH, _H), jnp.float32),
            pltpu.VMEM_SHARED((_N, _H), jnp.float32),
            pltpu.SemaphoreType.DMA,
        ],
    )


_RB = 1000
_NGRID = _N // _RB


def _dense1_body(x_ref, w_ref, dego_ref, out_ref):
    ns = lax.rsqrt(jnp.maximum(dego_ref[0] + dego_ref[1], 1.0))
    out_ref[...] = jnp.dot(x_ref[...], w_ref[...],
                           preferred_element_type=jnp.float32) * ns


def _dense1(x, W, dego_p):
    return pl.pallas_call(
        _dense1_body,
        grid=(_NGRID,),
        in_specs=[
            pl.BlockSpec((_RB, _D), lambda j: (j, 0)),
            pl.BlockSpec((_D, _H), lambda j: (0, 0)),
            pl.BlockSpec((2, _RB, 1), lambda j: (0, j, 0)),
        ],
        out_specs=pl.BlockSpec((_RB, _H), lambda j: (j, 0)),
        out_shape=jax.ShapeDtypeStruct((_N, _H), jnp.float32),
    )(x, W, dego_p)


def _dense2_body(p_ref, degi_ref, dego_ref, b1_ref, w_ref, out_ref):
    agg = p_ref[0] + p_ref[1]
    nd = lax.rsqrt(jnp.maximum(degi_ref[0] + degi_ref[1], 1.0))
    h = agg * nd + b1_ref[...]
    h = jnp.where(h > 0, h, jnp.exp(h) - 1.0)
    ns = lax.rsqrt(jnp.maximum(dego_ref[0] + dego_ref[1], 1.0))
    out_ref[...] = jnp.dot(h, w_ref[...],
                           preferred_element_type=jnp.float32) * ns


def _dense2(p, degi_p, dego_p, b1, W):
    return pl.pallas_call(
        _dense2_body,
        grid=(_NGRID,),
        in_specs=[
            pl.BlockSpec((2, _RB, _H), lambda j: (0, j, 0)),
            pl.BlockSpec((2, _RB, 1), lambda j: (0, j, 0)),
            pl.BlockSpec((2, _RB, 1), lambda j: (0, j, 0)),
            pl.BlockSpec((1, _H), lambda j: (0, 0)),
            pl.BlockSpec((_H, _H), lambda j: (0, 0)),
        ],
        out_specs=pl.BlockSpec((_RB, _H), lambda j: (j, 0)),
        out_shape=jax.ShapeDtypeStruct((_N, _H), jnp.float32),
    )(p, degi_p, dego_p, b1, W)


def _final_body(p_ref, degi_ref, b2_ref, wc_ref, bc_ref, out_ref, acc_ref):
    j = pl.program_id(0)

    @pl.when(j == 0)
    def _():
        acc_ref[...] = jnp.zeros_like(acc_ref)

    agg = p_ref[0] + p_ref[1]
    nd = lax.rsqrt(jnp.maximum(degi_ref[0] + degi_ref[1], 1.0))
    z = agg * nd + b2_ref[...]
    z = jnp.where(z > 0, z, jnp.exp(z) - 1.0)
    acc_ref[...] += jnp.sum(z, axis=0, keepdims=True)

    @pl.when(j == pl.num_programs(0) - 1)
    def _():
        out_ref[...] = jnp.dot(acc_ref[...] * (1.0 / _N), wc_ref[...],
                               preferred_element_type=jnp.float32) + bc_ref[...]


def _final(p, degi_p, b2, Wc, bc):
    return pl.pallas_call(
        _final_body,
        grid=(_NGRID,),
        in_specs=[
            pl.BlockSpec((2, _RB, _H), lambda j: (0, j, 0)),
            pl.BlockSpec((2, _RB, 1), lambda j: (0, j, 0)),
            pl.BlockSpec((1, _H), lambda j: (0, 0)),
            pl.BlockSpec((_H, _C), lambda j: (0, 0)),
            pl.BlockSpec((1, _C), lambda j: (0, 0)),
        ],
        out_specs=pl.BlockSpec((1, _C), lambda j: (0, 0)),
        out_shape=jax.ShapeDtypeStruct((1, _C), jnp.float32),
        scratch_shapes=[pltpu.VMEM((1, _H), jnp.float32)],
    )(p, degi_p, b2, Wc, bc)


def kernel(features, edge_index, W1, b1, W2, b2, Wc, bc):
    src = edge_index[0]
    dst = edge_index[1]

    lane = jnp.arange(_H, dtype=jnp.int32)
    ones = jnp.stack([
        jnp.where(lane < _H // 2, 1.0, 0.0),
        jnp.where(lane >= _H // 2, 1.0, 0.0),
    ]).astype(jnp.float32)[:, None, :] * jnp.ones((2, _CH, _H), jnp.float32)
    zeros_h = jnp.zeros((_N, _H), jnp.float32)

    degp = _deg_call()(src, dst, ones, zeros_h)
    dego_p = degp[:, :, 0:1]
    degi_p = degp[:, :, _H - 1:_H]

    b1r = b1.reshape(1, _H)
    b2r = b2.reshape(1, _H)
    bcr = bc.reshape(1, _C)

    hs1 = _dense1(features, W1, dego_p)
    p1 = _mp_call()(hs1, src, dst, zeros_h)
    hs2 = _dense2(p1, degi_p, dego_p, b1r, W2)
    p2 = _mp_call()(hs2, src, dst, zeros_h)
    return _final(p2, degi_p, b2r, Wc, bcr)


# speedup vs baseline: 5.9221x; 1.3738x over previous
import functools

import jax
import jax.numpy as jnp
from jax import lax
from jax.experimental import pallas as pl
from jax.experimental.pallas import tpu as pltpu
from jax.experimental.pallas import tpu_sc as plsc

_N = 10000
_E = 320000
_D = 128
_H = 128
_C = 64

_NC = 2
_NS = 16
_NW = _NC * _NS
_EPW = _E // _NW
_CH = 40
_NCHUNK = _EPW // _CH
_RPT = 624
_TAIL0 = _RPT * _NS
_TAILN = _N - _TAIL0
_DEGW = 16

def _deg_body(src_hbm, dst_hbm, ones_hbm, zeros_hbm, out_hbm,
              src_v, dst_v, src_v2, dst_v2, ones_v, acc,
              ix0, ix1, ss0, ss1):
    cid = lax.axis_index("c")
    sid = lax.axis_index("s")
    wid = sid * _NC + cid
    row0 = sid * _RPT
    pltpu.sync_copy(zeros_hbm.at[pl.ds(row0, _RPT)], acc.at[pl.ds(row0, _RPT)])

    @pl.when(sid == 0)
    def _():
        pltpu.sync_copy(zeros_hbm.at[pl.ds(_TAIL0, _TAILN)], acc.at[pl.ds(_TAIL0, _TAILN)])

    pltpu.sync_copy(ones_hbm, ones_v)
    plsc.subcore_barrier()

    base0 = wid * _EPW

    def idx_start(i, sv, dv, sem):
        base = base0 + i * _CH
        pltpu.async_copy(src_hbm.at[pl.ds(base, _CH)], sv, sem)
        pltpu.async_copy(dst_hbm.at[pl.ds(base, _CH)], dv, sem)

    def idx_wait(i, sv, dv, sem):
        base = base0 + i * _CH
        pltpu.make_async_copy(src_hbm.at[pl.ds(base, _CH)], sv, sem).wait()
        pltpu.make_async_copy(dst_hbm.at[pl.ds(base, _CH)], dv, sem).wait()

    def pair_wait(sv, dv, sem):
        pltpu.make_async_copy(ones_v.at[0], acc.at[sv], sem).wait()
        pltpu.make_async_copy(ones_v.at[1], acc.at[dv], sem).wait()

    idx_start(0, src_v, dst_v, ix0)
    idx_start(1, src_v2, dst_v2, ix1)

    def half(i, sv, dv, ixs, ss, nsv, ndv, nixs, ns):
        idx_wait(i, sv, dv, ixs)
        pltpu.async_copy(ones_v.at[0], acc.at[sv], ss, add=True)
        pltpu.async_copy(ones_v.at[1], acc.at[dv], ss, add=True)

        @pl.when(i >= 1)
        def _():
            pair_wait(nsv, ndv, ns)

            @pl.when(i + 1 < _NCHUNK)
            def _():
                idx_start(i + 1, nsv, ndv, nixs)

    def body(k, carry):
        i0 = 2 * k
        half(i0, src_v, dst_v, ix0, ss0, src_v2, dst_v2, ix1, ss1)
        half(i0 + 1, src_v2, dst_v2, ix1, ss1, src_v, dst_v, ix0, ss0)
        return carry

    lax.fori_loop(0, _NCHUNK // 2, body, 0)
    pair_wait(src_v2, dst_v2, ss1)
    plsc.subcore_barrier()
    pltpu.sync_copy(acc.at[pl.ds(row0, _RPT)], out_hbm.at[cid, pl.ds(row0, _RPT)])

    @pl.when(sid == 0)
    def _():
        pltpu.sync_copy(acc.at[pl.ds(_TAIL0, _TAILN)], out_hbm.at[cid, pl.ds(_TAIL0, _TAILN)])


@functools.cache
def _deg_call():
    mesh = plsc.VectorSubcoreMesh(
        core_axis_name="c", subcore_axis_name="s",
        num_cores=_NC, num_subcores=_NS,
    )
    return pl.kernel(
        _deg_body,
        out_type=jax.ShapeDtypeStruct((_NC, _N, _H), jnp.float32),
        mesh=mesh,
        scratch_types=[
            pltpu.VMEM((_CH,), jnp.int32),
            pltpu.VMEM((_CH,), jnp.int32),
            pltpu.VMEM((_CH,), jnp.int32),
            pltpu.VMEM((_CH,), jnp.int32),
            pltpu.VMEM((2, _CH, _H), jnp.float32),
            pltpu.VMEM_SHARED((_N, _H), jnp.float32),
            pltpu.SemaphoreType.DMA,
            pltpu.SemaphoreType.DMA,
            pltpu.SemaphoreType.DMA,
            pltpu.SemaphoreType.DMA,
        ],
    )


def _mp_body(h_hbm, src_hbm, dst_hbm, zeros_hbm, out_hbm,
             src0, src1, dst0, dst1, rows0, rows1, acc,
             ix0, ix1, g0, g1, s0, s1):
    cid = lax.axis_index("c")
    sid = lax.axis_index("s")
    wid = sid * _NC + cid
    row0 = sid * _RPT
    pltpu.sync_copy(zeros_hbm.at[pl.ds(row0, _RPT)], acc.at[pl.ds(row0, _RPT)])

    @pl.when(sid == 0)
    def _():
        pltpu.sync_copy(zeros_hbm.at[pl.ds(_TAIL0, _TAILN)], acc.at[pl.ds(_TAIL0, _TAILN)])

    plsc.subcore_barrier()

    base0 = wid * _EPW

    def idx_start(i, sv, dv, sem):
        base = base0 + i * _CH
        pltpu.async_copy(src_hbm.at[pl.ds(base, _CH)], sv, sem)
        pltpu.async_copy(dst_hbm.at[pl.ds(base, _CH)], dv, sem)

    def idx_wait(i, sv, dv, sem):
        base = base0 + i * _CH
        pltpu.make_async_copy(src_hbm.at[pl.ds(base, _CH)], sv, sem).wait()
        pltpu.make_async_copy(dst_hbm.at[pl.ds(base, _CH)], dv, sem).wait()

    idx_start(0, src0, dst0, ix0)
    idx_start(1, src1, dst1, ix1)
    idx_wait(0, src0, dst0, ix0)
    pltpu.async_copy(h_hbm.at[src0], rows0, g0)

    def half(i, sv, dv, ixs, rows_b, g_b, s_b,
             nsv, ndv, nixs, nrows, ng, ns):
        pltpu.make_async_copy(h_hbm.at[sv], rows_b, g_b).wait()
        pltpu.async_copy(rows_b, acc.at[dv], s_b, add=True)

        @pl.when(i + 1 < _NCHUNK)
        def _():
            idx_wait(i + 1, nsv, ndv, nixs)
            pltpu.async_copy(h_hbm.at[nsv], nrows, ng)

        @pl.when(i + 2 < _NCHUNK)
        def _():
            pltpu.make_async_copy(rows_b, acc.at[dv], s_b).wait()
            idx_start(i + 2, sv, dv, ixs)

    def body(k, carry):
        i0 = 2 * k
        half(i0, src0, dst0, ix0, rows0, g0, s0,
             src1, dst1, ix1, rows1, g1, s1)
        half(i0 + 1, src1, dst1, ix1, rows1, g1, s1,
             src0, dst0, ix0, rows0, g0, s0)
        return carry

    lax.fori_loop(0, _NCHUNK // 2, body, 0)
    pltpu.make_async_copy(rows0, acc.at[dst0], s0).wait()
    pltpu.make_async_copy(rows1, acc.at[dst1], s1).wait()
    plsc.subcore_barrier()
    pltpu.sync_copy(acc.at[pl.ds(row0, _RPT)], out_hbm.at[cid, pl.ds(row0, _RPT)])

    @pl.when(sid == 0)
    def _():
        pltpu.sync_copy(acc.at[pl.ds(_TAIL0, _TAILN)], out_hbm.at[cid, pl.ds(_TAIL0, _TAILN)])


@functools.cache
def _mp_call():
    mesh = plsc.VectorSubcoreMesh(
        core_axis_name="c", subcore_axis_name="s",
        num_cores=_NC, num_subcores=_NS,
    )
    return pl.kernel(
        _mp_body,
        out_type=jax.ShapeDtypeStruct((_NC, _N, _H), jnp.float32),
        mesh=mesh,
        scratch_types=[
            pltpu.VMEM((_CH,), jnp.int32),
            pltpu.VMEM((_CH,), jnp.int32),
            pltpu.VMEM((_CH,), jnp.int32),
            pltpu.VMEM((_CH,), jnp.int32),
            pltpu.VMEM((_CH, _H), jnp.float32),
            pltpu.VMEM((_CH, _H), jnp.float32),
            pltpu.VMEM_SHARED((_N, _H), jnp.float32),
            pltpu.SemaphoreType.DMA,
            pltpu.SemaphoreType.DMA,
            pltpu.SemaphoreType.DMA,
            pltpu.SemaphoreType.DMA,
            pltpu.SemaphoreType.DMA,
            pltpu.SemaphoreType.DMA,
        ],
    )


_RB = 1000
_NGRID = _N // _RB


def _dense1_body(x_ref, w_ref, dego_ref, out_ref):
    ns = lax.rsqrt(jnp.maximum(dego_ref[0] + dego_ref[1], 1.0))
    out_ref[...] = jnp.dot(x_ref[...], w_ref[...],
                           preferred_element_type=jnp.float32) * ns


def _dense1(x, W, dego_p):
    return pl.pallas_call(
        _dense1_body,
        grid=(_NGRID,),
        in_specs=[
            pl.BlockSpec((_RB, _D), lambda j: (j, 0)),
            pl.BlockSpec((_D, _H), lambda j: (0, 0)),
            pl.BlockSpec((2, _RB, 1), lambda j: (0, j, 0)),
        ],
        out_specs=pl.BlockSpec((_RB, _H), lambda j: (j, 0)),
        out_shape=jax.ShapeDtypeStruct((_N, _H), jnp.float32),
    )(x, W, dego_p)


def _dense2_body(p_ref, degi_ref, dego_ref, b1_ref, w_ref, out_ref):
    agg = p_ref[0] + p_ref[1]
    nd = lax.rsqrt(jnp.maximum(degi_ref[0] + degi_ref[1], 1.0))
    h = agg * nd + b1_ref[...]
    h = jnp.where(h > 0, h, jnp.exp(h) - 1.0)
    ns = lax.rsqrt(jnp.maximum(dego_ref[0] + dego_ref[1], 1.0))
    out_ref[...] = jnp.dot(h, w_ref[...],
                           preferred_element_type=jnp.float32) * ns


def _dense2(p, degi_p, dego_p, b1, W):
    return pl.pallas_call(
        _dense2_body,
        grid=(_NGRID,),
        in_specs=[
            pl.BlockSpec((2, _RB, _H), lambda j: (0, j, 0)),
            pl.BlockSpec((2, _RB, 1), lambda j: (0, j, 0)),
            pl.BlockSpec((2, _RB, 1), lambda j: (0, j, 0)),
            pl.BlockSpec((1, _H), lambda j: (0, 0)),
            pl.BlockSpec((_H, _H), lambda j: (0, 0)),
        ],
        out_specs=pl.BlockSpec((_RB, _H), lambda j: (j, 0)),
        out_shape=jax.ShapeDtypeStruct((_N, _H), jnp.float32),
    )(p, degi_p, dego_p, b1, W)


def _final_body(p_ref, degi_ref, b2_ref, wc_ref, bc_ref, out_ref, acc_ref):
    j = pl.program_id(0)

    @pl.when(j == 0)
    def _():
        acc_ref[...] = jnp.zeros_like(acc_ref)

    agg = p_ref[0] + p_ref[1]
    nd = lax.rsqrt(jnp.maximum(degi_ref[0] + degi_ref[1], 1.0))
    z = agg * nd + b2_ref[...]
    z = jnp.where(z > 0, z, jnp.exp(z) - 1.0)
    acc_ref[...] += jnp.sum(z, axis=0, keepdims=True)

    @pl.when(j == pl.num_programs(0) - 1)
    def _():
        out_ref[...] = jnp.dot(acc_ref[...] * (1.0 / _N), wc_ref[...],
                               preferred_element_type=jnp.float32) + bc_ref[...]


def _final(p, degi_p, b2, Wc, bc):
    return pl.pallas_call(
        _final_body,
        grid=(_NGRID,),
        in_specs=[
            pl.BlockSpec((2, _RB, _H), lambda j: (0, j, 0)),
            pl.BlockSpec((2, _RB, 1), lambda j: (0, j, 0)),
            pl.BlockSpec((1, _H), lambda j: (0, 0)),
            pl.BlockSpec((_H, _C), lambda j: (0, 0)),
            pl.BlockSpec((1, _C), lambda j: (0, 0)),
        ],
        out_specs=pl.BlockSpec((1, _C), lambda j: (0, 0)),
        out_shape=jax.ShapeDtypeStruct((1, _C), jnp.float32),
        scratch_shapes=[pltpu.VMEM((1, _H), jnp.float32)],
    )(p, degi_p, b2, Wc, bc)


def kernel(features, edge_index, W1, b1, W2, b2, Wc, bc):
    src = edge_index[0]
    dst = edge_index[1]

    lane = jnp.arange(_H, dtype=jnp.int32)
    ones = jnp.stack([
        jnp.where(lane < _H // 2, 1.0, 0.0),
        jnp.where(lane >= _H // 2, 1.0, 0.0),
    ]).astype(jnp.float32)[:, None, :] * jnp.ones((2, _CH, _H), jnp.float32)
    zeros_h = jnp.zeros((_N, _H), jnp.float32)

    degp = _deg_call()(src, dst, ones, zeros_h)
    dego_p = degp[:, :, 0:1]
    degi_p = degp[:, :, _H - 1:_H]

    b1r = b1.reshape(1, _H)
    b2r = b2.reshape(1, _H)
    bcr = bc.reshape(1, _C)

    hs1 = _dense1(features, W1, dego_p)
    p1 = _mp_call()(hs1, src, dst, zeros_h)
    hs2 = _dense2(p1, degi_p, dego_p, b1r, W2)
    p2 = _mp_call()(hs2, src, dst, zeros_h)
    return _final(p2, degi_p, b2r, Wc, bcr)
